# baseline (speedup 1.0000x reference)
import functools

import jax
import jax.numpy as jnp
from jax import lax
from jax.experimental import pallas as pl
from jax.experimental.pallas import tpu as pltpu
from jax.experimental.pallas import tpu_sc as plsc

_BATCH = 16384
_HIST = 50
_DIM = 64
_NB = _BATCH * _HIST
_VOCAB = 1000000

_INFO = plsc.get_sparse_core_info()
_NC = _INFO.num_cores
_NS = _INFO.num_subcores
_NW = _NC * _NS

_VB = 512
_BLK_PER_W = 61
_V_EXTRA = _BLK_PER_W * _NW * _VB
_V_TAIL = _V_EXTRA + _VB
_TAILW = _VOCAB - _V_TAIL

_ROWS_PER_W = _NB // _NW
_CHUNK = 128
_K = 5
_GROUP = _CHUNK * _K
_NGRP = _ROWS_PER_W // _GROUP
_NBUF = 2


def _xpose_kernel(tbl_hbm, tail_hbm, scr_hbm, tbuf, tbufT):
    wid = lax.axis_index("s") * _NC + lax.axis_index("c")
    lanes = lax.iota(jnp.int32, 16) * _DIM

    def transpose_cols(width):
        def vg_body(vg):
            base = lanes + vg * 1024
            for c in range(_DIM):
                x = tbuf[c, pl.ds(vg * 16, 16)]
                plsc.store_scatter(tbufT, [base + c], x)
        pl.loop(0, width // 16)(vg_body)

    def do_block(v0):
        pltpu.sync_copy(tbl_hbm.at[:, pl.ds(v0, _VB)], tbuf)
        transpose_cols(_VB)
        pltpu.sync_copy(tbufT, scr_hbm.at[pl.ds(v0 * _DIM, _VB * _DIM)])

    def body(k):
        do_block((wid * _BLK_PER_W + k) * _VB)

    pl.loop(0, _BLK_PER_W)(body)

    @pl.when(wid == 0)
    def _():
        do_block(_V_EXTRA)

    @pl.when(wid == 1)
    def _():
        pltpu.sync_copy(tail_hbm, tbufT.at[pl.ds(0, _TAILW * _DIM)])
        pltpu.sync_copy(tbufT.at[pl.ds(0, _TAILW * _DIM)],
                        scr_hbm.at[pl.ds(_V_TAIL * _DIM, _TAILW * _DIM)])


def _embed_kernel(idx_hbm, table_hbm, out_hbm, idx_v, rows0, rows1, gsem,
                  wsem0, wsem1):
    wid = lax.axis_index("s") * _NC + lax.axis_index("c")
    wbase = wid * _ROWS_PER_W
    rows = (rows0, rows1)
    wsems = (wsem0, wsem1)

    pltpu.sync_copy(idx_hbm.at[pl.ds(wbase, _ROWS_PER_W)], idx_v)

    def fire_and_drain(g, buf):
        copies = [
            pltpu.async_copy(
                table_hbm.at[idx_v.at[pl.ds(g * _GROUP + j * _CHUNK, _CHUNK)]],
                buf.at[pl.ds(j * _CHUNK, _CHUNK)],
                gsem,
            )
            for j in range(_K)
        ]
        for c in copies:
            c.wait()

    def start_writeback(g, buf, sem):
        pltpu.async_copy(buf, out_hbm.at[pl.ds(wbase + g * _GROUP, _GROUP)], sem)

    for b in range(_NBUF):
        fire_and_drain(b, rows[b])
        start_writeback(b, rows[b], wsems[b])

    def body(gg):
        for b in range(_NBUF):
            g = gg + b
            pltpu.make_async_copy(
                rows[b], out_hbm.at[pl.ds(wbase, _GROUP)], wsems[b]).wait()
            fire_and_drain(g, rows[b])
            start_writeback(g, rows[b], wsems[b])

    pl.loop(_NBUF, _NGRP, step=_NBUF)(body)

    for b in range(_NBUF):
        pltpu.make_async_copy(
            rows[b], out_hbm.at[pl.ds(wbase, _GROUP)], wsems[b]).wait()


@jax.jit
def _run(indices_flat, table_t, tail_flat):
    mesh = plsc.VectorSubcoreMesh(core_axis_name="c", subcore_axis_name="s")
    xpose = functools.partial(
        pl.kernel,
        mesh=mesh,
        out_type=jax.ShapeDtypeStruct((_VOCAB * _DIM,), jnp.float32),
        scratch_types=[
            pltpu.VMEM((_DIM, _VB), jnp.float32),
            pltpu.VMEM((_VB * _DIM,), jnp.float32),
        ],
        compiler_params=pltpu.CompilerParams(use_tc_tiling_on_sc=True,
                                             needs_layout_passes=False),
    )(_xpose_kernel)
    scr = xpose(table_t, tail_flat)
    embed = functools.partial(
        pl.kernel,
        mesh=mesh,
        out_type=jax.ShapeDtypeStruct((_NB, _DIM), jnp.float32),
        scratch_types=[
            pltpu.VMEM((_ROWS_PER_W,), jnp.int32),
            pltpu.VMEM((_GROUP, _DIM), jnp.float32),
            pltpu.VMEM((_GROUP, _DIM), jnp.float32),
            pltpu.SemaphoreType.DMA,
            pltpu.SemaphoreType.DMA,
            pltpu.SemaphoreType.DMA,
        ],
        compiler_params=pltpu.CompilerParams(use_tc_tiling_on_sc=False),
    )(_embed_kernel)
    return embed(indices_flat, scr.reshape(_VOCAB, _DIM))


def kernel(indices, table):
    out = _run(indices.reshape(-1), jnp.transpose(table),
               table[_V_TAIL:].reshape(-1))
    return out.reshape(*indices.shape, table.shape[1])


# speedup vs baseline: 1.1757x; 1.1757x over previous
import functools

import jax
import jax.numpy as jnp
from jax import lax
from jax.experimental import pallas as pl
from jax.experimental.pallas import tpu as pltpu
from jax.experimental.pallas import tpu_sc as plsc

_BATCH = 16384
_HIST = 50
_DIM = 64
_NB = _BATCH * _HIST
_VOCAB = 1000000

_INFO = plsc.get_sparse_core_info()
_NC = _INFO.num_cores
_NS = _INFO.num_subcores
_NW = _NC * _NS

_VB = 512
_BLK_PER_W = 61
_V_EXTRA = _BLK_PER_W * _NW * _VB
_V_TAIL = _V_EXTRA + _VB
_TAILW = _VOCAB - _V_TAIL

_ROWS_PER_W = _NB // _NW
_CHUNK = 128
_K = 5
_GROUP = _CHUNK * _K
_NGRP = _ROWS_PER_W // _GROUP
_NBUF = 2


def _xpose_kernel(tbl_hbm, tail_hbm, scr_hbm, tbuf, tbufT):
    wid = lax.axis_index("s") * _NC + lax.axis_index("c")
    lanes = lax.iota(jnp.int32, 16) * _DIM

    def transpose_cols(width):
        def vg_body(vg):
            base = lanes + vg * 1024
            for c0 in range(0, _DIM, 16):
                xs = [tbuf[c0 + i, pl.ds(vg * 16, 16)] for i in range(16)]
                for i in range(16):
                    plsc.store_scatter(tbufT, [base + (c0 + i)], xs[i])
        plsc.parallel_loop(0, width // 16, unroll=1)(vg_body)

    def do_block(v0):
        pltpu.sync_copy(tbl_hbm.at[:, pl.ds(v0, _VB)], tbuf)
        transpose_cols(_VB)
        pltpu.sync_copy(tbufT, scr_hbm.at[pl.ds(v0 * _DIM, _VB * _DIM)])

    def body(k):
        do_block((wid * _BLK_PER_W + k) * _VB)

    pl.loop(0, _BLK_PER_W)(body)

    @pl.when(wid == 0)
    def _():
        do_block(_V_EXTRA)

    @pl.when(wid == 1)
    def _():
        pltpu.sync_copy(tail_hbm, tbufT.at[pl.ds(0, _TAILW * _DIM)])
        pltpu.sync_copy(tbufT.at[pl.ds(0, _TAILW * _DIM)],
                        scr_hbm.at[pl.ds(_V_TAIL * _DIM, _TAILW * _DIM)])


def _embed_kernel(idx_hbm, table_hbm, out_hbm, idx_v, rows0, rows1, gsem,
                  wsem0, wsem1):
    wid = lax.axis_index("s") * _NC + lax.axis_index("c")
    wbase = wid * _ROWS_PER_W
    rows = (rows0, rows1)
    wsems = (wsem0, wsem1)

    pltpu.sync_copy(idx_hbm.at[pl.ds(wbase, _ROWS_PER_W)], idx_v)

    def fire_and_drain(g, buf):
        copies = [
            pltpu.async_copy(
                table_hbm.at[idx_v.at[pl.ds(g * _GROUP + j * _CHUNK, _CHUNK)]],
                buf.at[pl.ds(j * _CHUNK, _CHUNK)],
                gsem,
            )
            for j in range(_K)
        ]
        for c in copies:
            c.wait()

    def start_writeback(g, buf, sem):
        pltpu.async_copy(buf, out_hbm.at[pl.ds(wbase + g * _GROUP, _GROUP)], sem)

    for b in range(_NBUF):
        fire_and_drain(b, rows[b])
        start_writeback(b, rows[b], wsems[b])

    def body(gg):
        for b in range(_NBUF):
            g = gg + b
            pltpu.make_async_copy(
                rows[b], out_hbm.at[pl.ds(wbase, _GROUP)], wsems[b]).wait()
            fire_and_drain(g, rows[b])
            start_writeback(g, rows[b], wsems[b])

    pl.loop(_NBUF, _NGRP, step=_NBUF)(body)

    for b in range(_NBUF):
        pltpu.make_async_copy(
            rows[b], out_hbm.at[pl.ds(wbase, _GROUP)], wsems[b]).wait()


@jax.jit
def _run(indices_flat, table_t, tail_flat):
    mesh = plsc.VectorSubcoreMesh(core_axis_name="c", subcore_axis_name="s")
    xpose = functools.partial(
        pl.kernel,
        mesh=mesh,
        out_type=jax.ShapeDtypeStruct((_VOCAB * _DIM,), jnp.float32),
        scratch_types=[
            pltpu.VMEM((_DIM, _VB), jnp.float32),
            pltpu.VMEM((_VB * _DIM,), jnp.float32),
        ],
        compiler_params=pltpu.CompilerParams(use_tc_tiling_on_sc=True,
                                             needs_layout_passes=False,
                                             disable_bounds_checks=True),
    )(_xpose_kernel)
    scr = xpose(table_t, tail_flat)
    embed = functools.partial(
        pl.kernel,
        mesh=mesh,
        out_type=jax.ShapeDtypeStruct((_NB, _DIM), jnp.float32),
        scratch_types=[
            pltpu.VMEM((_ROWS_PER_W,), jnp.int32),
            pltpu.VMEM((_GROUP, _DIM), jnp.float32),
            pltpu.VMEM((_GROUP, _DIM), jnp.float32),
            pltpu.SemaphoreType.DMA,
            pltpu.SemaphoreType.DMA,
            pltpu.SemaphoreType.DMA,
        ],
        compiler_params=pltpu.CompilerParams(use_tc_tiling_on_sc=False),
    )(_embed_kernel)
    return embed(indices_flat, scr.reshape(_VOCAB, _DIM))


def kernel(indices, table):
    out = _run(indices.reshape(-1), jnp.transpose(table),
               table[_V_TAIL:].reshape(-1))
    return out.reshape(*indices.shape, table.shape[1])


# speedup vs baseline: 1.7314x; 1.4727x over previous
import functools

import jax
import jax.numpy as jnp
from jax import lax
from jax.experimental import pallas as pl
from jax.experimental.pallas import tpu as pltpu
from jax.experimental.pallas import tpu_sc as plsc

_BATCH = 16384
_HIST = 50
_DIM = 64
_PDIM = 128
_NB = _BATCH * _HIST
_VOCAB = 1000000

_INFO = plsc.get_sparse_core_info()
_NC = _INFO.num_cores
_NS = _INFO.num_subcores
_NW = _NC * _NS

_CHUNK = 128
_TBB = _BATCH // _CHUNK
_NBLK = _NB // _CHUNK
_BLK_PER_W = _NBLK // _NW
_ROWS_PER_W = _NB // _NW


def _lookup_kernel(idx_hbm, tbl_hbm, out_hbm, ibuf, gb0, gb1, ob0, ob1,
                   gs0, gs1, os0, os1):
    wid = lax.axis_index("s") * _NC + lax.axis_index("c")
    wbase = wid * _ROWS_PER_W
    gbase = wid * _BLK_PER_W
    gbs = (gb0, gb1)
    obs = (ob0, ob1)
    gss = (gs0, gs1)
    oss = (os0, os1)
    lanes = lax.iota(jnp.int32, 16)

    pltpu.sync_copy(idx_hbm.at[pl.ds(wbase, _ROWS_PER_W)], ibuf)

    def fire_gather(k, buf, sem):
        pltpu.async_copy(tbl_hbm.at[ibuf.at[pl.ds(k * _CHUNK, _CHUNK)]],
                         buf, sem)

    def absorb_gather(buf, sem):
        pltpu.make_async_copy(tbl_hbm.at[pl.ds(0, _CHUNK)], buf, sem).wait()

    def absorb_wb(buf, sem):
        pltpu.make_async_copy(buf, out_hbm.at[0, :, pl.ds(0, _CHUNK)],
                              sem).wait()

    def transpose(gb, ob):
        def lg_body(lg):
            lvec = lanes + lg * 16
            for c0 in range(0, _DIM, 16):
                xs = [
                    plsc.load_gather(
                        gb, [lvec, jnp.full((16,), c0 + i, jnp.int32)])
                    for i in range(16)
                ]
                for i in range(16):
                    ob[c0 + i, pl.ds(lg * 16, 16)] = xs[i]
        plsc.parallel_loop(0, _CHUNK // 16, unroll=1)(lg_body)

    def writeback(g, ob, sem):
        h = g // _TBB
        tb = g % _TBB
        pltpu.async_copy(ob, out_hbm.at[h, :, pl.ds(tb * _CHUNK, _CHUNK)],
                         sem)

    fire_gather(0, gbs[0], gss[0])
    fire_gather(1, gbs[1], gss[1])

    def body(kk):
        for b in range(2):
            k = kk + b
            absorb_gather(gbs[b], gss[b])

            @pl.when(kk >= 2)
            def _():
                absorb_wb(obs[b], oss[b])

            transpose(gbs[b], obs[b])
            writeback(gbase + k, obs[b], oss[b])

            @pl.when(kk < _BLK_PER_W - 2)
            def _():
                fire_gather(k + 2, gbs[b], gss[b])

    pl.loop(0, _BLK_PER_W, step=2)(body)

    absorb_wb(obs[0], oss[0])
    absorb_wb(obs[1], oss[1])


@jax.jit
def _run(indices, table):
    padded = jnp.pad(table, ((0, 0), (0, _PDIM - _DIM)))
    idx_hm = jnp.transpose(indices).reshape(-1)
    mesh = plsc.VectorSubcoreMesh(core_axis_name="c", subcore_axis_name="s")
    lookup = functools.partial(
        pl.kernel,
        mesh=mesh,
        out_type=jax.ShapeDtypeStruct((_HIST, _DIM, _BATCH), jnp.float32),
        scratch_types=[
            pltpu.VMEM((_ROWS_PER_W,), jnp.int32),
            pltpu.VMEM((_CHUNK, _PDIM), jnp.float32),
            pltpu.VMEM((_CHUNK, _PDIM), jnp.float32),
            pltpu.VMEM((_DIM, _CHUNK), jnp.float32),
            pltpu.VMEM((_DIM, _CHUNK), jnp.float32),
            pltpu.SemaphoreType.DMA,
            pltpu.SemaphoreType.DMA,
            pltpu.SemaphoreType.DMA,
            pltpu.SemaphoreType.DMA,
        ],
        compiler_params=pltpu.CompilerParams(use_tc_tiling_on_sc=True,
                                             needs_layout_passes=False,
                                             disable_bounds_checks=True),
    )(_lookup_kernel)
    out3 = lookup(idx_hm, padded)
    return jnp.transpose(out3, (2, 0, 1))


def kernel(indices, table):
    return _run(indices, table)
